# baseline (speedup 1.0000x reference)
import functools

import jax
import jax.numpy as jnp
from jax import lax
from jax.experimental import pallas as pl
from jax.experimental.pallas import tpu as pltpu
from jax.experimental.pallas import tpu_sc as plsc

ALPHA = 2.0
NEG_WEIGHT = 1.0
MIN_IOU = 0.05

L = 16
ROWS = 16
B, M, N = 32, 1024, 1024
S_SC = 12
NCHUNK = M // ROWS
NJC = N // L
NPAIR = NCHUNK // 2
R_TC = 64


def _sc_body(scores_hbm, ious_hbm, out_hbm,
             iou0, iou1, sc0, sc1, lastrow, colsum, rowvec, out_buf,
             sem_i0, sem_i1, sem_s0, sem_s1):
    cid = lax.axis_index("c")
    sid = lax.axis_index("s")
    b = sid * 2 + cid

    @pl.when(b < S_SC)
    def _():
        def iou_copy(g, buf, sem):
            return pltpu.make_async_copy(
                ious_hbm.at[b, pl.ds(g * ROWS, ROWS), pl.ds(0, N)], buf, sem)

        def sc_copy(g, buf, sem):
            return pltpu.make_async_copy(
                scores_hbm.at[b, pl.ds(g * ROWS, ROWS)], buf, sem)

        iou_copy(0, iou0, sem_i0).start()
        sc_copy(0, sc0, sem_s0).start()
        iou_copy(1, iou1, sem_i1).start()
        sc_copy(1, sc1, sem_s1).start()

        pltpu.sync_copy(scores_hbm.at[b, pl.ds(M, 1)], lastrow)

        def zbody(j, _):
            colsum[pl.ds(j * L, L)] = jnp.zeros((L,), jnp.float32)
            return 0
        lax.fori_loop(0, NJC, zbody, 0)

        zero_v = jnp.zeros((L,), jnp.float32)
        iota16 = jnp.arange(L, dtype=jnp.int32)

        def chunk_compute(g, iou_buf, sc_buf, s1, cnt, n0c, n0d):
            def make_jc_body(base):
                def jc_body(jc, c):
                    s1_, cnt_, raccs = c
                    off = jc * L
                    colacc = colsum[pl.ds(off, L)]
                    new_raccs = []
                    for i in range(ROWS // 2):
                        vio = iou_buf[base + i, pl.ds(off, L)]
                        vsc = sc_buf[base + i, pl.ds(off, L)]
                        m = vio >= MIN_IOU
                        t = jnp.where(m, jnp.minimum(vio, 1.0), 0.0)
                        s1_ = s1_ + vsc * t
                        cnt_ = cnt_ + jnp.where(m, 1.0, 0.0)
                        colacc = colacc + t
                        new_raccs.append(raccs[i] + t)
                    colsum[pl.ds(off, L)] = colacc
                    return s1_, cnt_, tuple(new_raccs)
                return jc_body

            for base in (0, ROWS // 2):
                s1, cnt, raccs = lax.fori_loop(
                    0, NJC, make_jc_body(base),
                    (s1, cnt, (zero_v,) * (ROWS // 2)))
                for i in range(ROWS // 2):
                    rowvec[base + i] = raccs[i]

            rowsum = plsc.load_gather(
                rowvec, [iota16, jnp.full((L,), 0, jnp.int32)])
            for c in range(1, L):
                rowsum = rowsum + plsc.load_gather(
                    rowvec, [iota16, jnp.full((L,), c, jnp.int32)])
            fvec = jnp.where(rowsum <= 0.001, 1.0, 0.0)
            lc = plsc.load_gather(
                sc_buf, [iota16, jnp.full((L,), N, jnp.int32)])
            return s1, cnt, n0c + fvec, n0d + fvec * lc

        def pair_body(p, carry):
            s1, cnt, n0c, n0d = carry
            g0 = 2 * p
            iou_copy(g0, iou0, sem_i0).wait()
            sc_copy(g0, sc0, sem_s0).wait()
            s1, cnt, n0c, n0d = chunk_compute(g0, iou0, sc0, s1, cnt, n0c, n0d)
            ge = jnp.minimum(g0 + 2, NCHUNK - 1)
            iou_copy(ge, iou0, sem_i0).start()
            sc_copy(ge, sc0, sem_s0).start()

            g1 = 2 * p + 1
            iou_copy(g1, iou1, sem_i1).wait()
            sc_copy(g1, sc1, sem_s1).wait()
            s1, cnt, n0c, n0d = chunk_compute(g1, iou1, sc1, s1, cnt, n0c, n0d)
            go = jnp.minimum(g1 + 2, NCHUNK - 1)
            iou_copy(go, iou1, sem_i1).start()
            sc_copy(go, sc1, sem_s1).start()
            return s1, cnt, n0c, n0d

        zero_v4 = jnp.zeros((L,), jnp.float32)
        s1, cnt, n0c, n0d = lax.fori_loop(
            0, NPAIR, pair_body, (zero_v4, zero_v4, zero_v4, zero_v4))

        iou_copy(NCHUNK - 1, iou0, sem_i0).wait()
        sc_copy(NCHUNK - 1, sc0, sem_s0).wait()
        iou_copy(NCHUNK - 1, iou1, sem_i1).wait()
        sc_copy(NCHUNK - 1, sc1, sem_s1).wait()

        def neg1_body(jc, c):
            n1c_, n1d_ = c
            off = jc * L
            v = colsum[pl.ds(off, L)]
            lr = lastrow[0, pl.ds(off, L)]
            n1 = jnp.where(v <= 0.001, 1.0, 0.0)
            return n1c_ + n1, n1d_ + n1 * lr
        n1c, n1d = lax.fori_loop(0, NJC, neg1_body, (zero_v, zero_v))

        s1s = jnp.full((L,), jnp.sum(s1), jnp.float32)
        cnts = jnp.full((L,), jnp.sum(cnt), jnp.float32)
        n0cs = jnp.full((L,), jnp.sum(n0c), jnp.float32)
        n0ds = jnp.full((L,), jnp.sum(n0d), jnp.float32)
        n1cs = jnp.full((L,), jnp.sum(n1c), jnp.float32)
        n1ds = jnp.full((L,), jnp.sum(n1d), jnp.float32)

        nll_pos = -(ALPHA * s1s) / jnp.maximum(cnts, 1.0)
        nll_neg = (-n0ds - n1ds) / (
            jnp.maximum(n0cs, 1.0) + jnp.maximum(n1cs, 1.0))
        loss = (nll_pos + NEG_WEIGHT * nll_neg) * (1.0 / B)

        out_buf[...] = loss
        pltpu.sync_copy(out_buf, out_hbm.at[b])


def _tc_body(sc_ref, iou_ref, lr_ref, out_ref, colsum, accs):
    r = pl.program_id(1)
    nr = pl.num_programs(1)

    @pl.when(r == 0)
    def _():
        accs[0] = 0.0
        accs[1] = 0.0
        accs[2] = 0.0
        accs[3] = 0.0
        colsum[...] = jnp.zeros_like(colsum)

    sc = sc_ref[0]
    io = iou_ref[0]
    m = io >= MIN_IOU
    t = jnp.where(m, jnp.minimum(io, 1.0), 0.0)
    accs[0] = accs[0] + jnp.sum(sc[:, :N] * t)
    accs[1] = accs[1] + jnp.sum(m.astype(jnp.float32))
    rowsum = jnp.sum(t, axis=1, keepdims=True)
    f = jnp.where(rowsum <= 0.001, 1.0, 0.0)
    accs[2] = accs[2] + jnp.sum(f)
    accs[3] = accs[3] + jnp.sum(f * sc[:, N:N + 1])
    colsum[...] = colsum[...] + jnp.sum(t, axis=0, keepdims=True)

    @pl.when(r == nr - 1)
    def _():
        cs = colsum[...]
        n1 = jnp.where(cs <= 0.001, 1.0, 0.0)
        lr = lr_ref[0, :, :N]
        n1c = jnp.sum(n1)
        n1d = jnp.sum(n1 * lr)
        nll_pos = -(ALPHA * accs[0]) / jnp.maximum(accs[1], 1.0)
        nll_neg = (-accs[3] - n1d) / (
            jnp.maximum(accs[2], 1.0) + jnp.maximum(n1c, 1.0))
        loss = (nll_pos + NEG_WEIGHT * nll_neg) * (1.0 / B)
        out_ref[...] = jnp.reshape(loss, (1, 1, 1))


@jax.jit
def _run(scores, ious):
    mesh = plsc.VectorSubcoreMesh(core_axis_name="c", subcore_axis_name="s")
    sc_fn = pl.kernel(
        _sc_body,
        out_type=jax.ShapeDtypeStruct((S_SC, L), jnp.float32),
        mesh=mesh,
        scratch_types=[
            pltpu.VMEM((ROWS, N), jnp.float32),
            pltpu.VMEM((ROWS, N), jnp.float32),
            pltpu.VMEM((ROWS, N + 1), jnp.float32),
            pltpu.VMEM((ROWS, N + 1), jnp.float32),
            pltpu.VMEM((1, N + 1), jnp.float32),
            pltpu.VMEM((N,), jnp.float32),
            pltpu.VMEM((ROWS, L), jnp.float32),
            pltpu.VMEM((L,), jnp.float32),
            pltpu.SemaphoreType.DMA,
            pltpu.SemaphoreType.DMA,
            pltpu.SemaphoreType.DMA,
            pltpu.SemaphoreType.DMA,
        ],
        compiler_params=pltpu.CompilerParams(
            use_tc_tiling_on_sc=True, needs_layout_passes=False),
    )
    sc_out = sc_fn(scores[:S_SC], ious)

    bt = B - S_SC
    scores_tc = scores[S_SC:]
    lastrow = scores[S_SC:, M, :].reshape(bt, 1, N + 1)
    tc_fn = pl.pallas_call(
        _tc_body,
        grid=(bt, M // R_TC),
        in_specs=[
            pl.BlockSpec((1, R_TC, N + 1), lambda s, r: (s, r, 0)),
            pl.BlockSpec((1, R_TC, N), lambda s, r: (s + S_SC, r, 0)),
            pl.BlockSpec((1, 1, N + 1), lambda s, r: (s, 0, 0)),
        ],
        out_specs=pl.BlockSpec((1, 1, 1), lambda s, r: (s, 0, 0)),
        out_shape=jax.ShapeDtypeStruct((bt, 1, 1), jnp.float32),
        scratch_shapes=[
            pltpu.VMEM((1, N), jnp.float32),
            pltpu.SMEM((4,), jnp.float32),
        ],
    )
    tc_out = tc_fn(scores_tc, ious, lastrow)

    return jnp.sum(sc_out[:, 0]) + jnp.sum(tc_out)


def kernel(logmax_scores, instance_ious, instance_matches):
    del instance_matches
    return _run(logmax_scores, instance_ious)


# speedup vs baseline: 1.3213x; 1.3213x over previous
import functools

import jax
import jax.numpy as jnp
from jax import lax
from jax.experimental import pallas as pl
from jax.experimental.pallas import tpu as pltpu
from jax.experimental.pallas import tpu_sc as plsc

ALPHA = 2.0
NEG_WEIGHT = 1.0
MIN_IOU = 0.05

L = 16
ROWS = 16
B, M, N = 32, 1024, 1024
S_SC = 12
NCHUNK = M // ROWS
NJC = N // L
NPAIR = NCHUNK // 2
R_TC = 256


def _sc_body(scores_hbm, ious_hbm, out_hbm,
             iou0, iou1, sc0, sc1, lastrow, colsum, rowvec, out_buf,
             sem_i0, sem_i1, sem_s0, sem_s1):
    cid = lax.axis_index("c")
    sid = lax.axis_index("s")
    b = sid * 2 + cid

    @pl.when(b < S_SC)
    def _():
        def iou_copy(g, buf, sem):
            return pltpu.make_async_copy(
                ious_hbm.at[b, pl.ds(g * ROWS, ROWS), pl.ds(0, N)], buf, sem)

        def sc_copy(g, buf, sem):
            return pltpu.make_async_copy(
                scores_hbm.at[b, pl.ds(g * ROWS, ROWS)], buf, sem)

        iou_copy(0, iou0, sem_i0).start()
        sc_copy(0, sc0, sem_s0).start()
        iou_copy(1, iou1, sem_i1).start()
        sc_copy(1, sc1, sem_s1).start()

        pltpu.sync_copy(scores_hbm.at[b, pl.ds(M, 1)], lastrow)

        def zbody(j, _):
            colsum[pl.ds(j * L, L)] = jnp.zeros((L,), jnp.float32)
            return 0
        lax.fori_loop(0, NJC, zbody, 0)

        zero_v = jnp.zeros((L,), jnp.float32)
        iota16 = jnp.arange(L, dtype=jnp.int32)

        def chunk_compute(g, iou_buf, sc_buf, s1, cnt, n0c, n0d):
            def make_jc_body(base):
                def jc_body(jc, c):
                    s1_, cnt_, raccs = c
                    off = jc * L
                    colacc = colsum[pl.ds(off, L)]
                    new_raccs = []
                    for i in range(ROWS // 2):
                        vio = iou_buf[base + i, pl.ds(off, L)]
                        vsc = sc_buf[base + i, pl.ds(off, L)]
                        m = vio >= MIN_IOU
                        t = jnp.where(m, jnp.minimum(vio, 1.0), 0.0)
                        s1_ = s1_ + vsc * t
                        cnt_ = cnt_ + jnp.where(m, 1.0, 0.0)
                        colacc = colacc + t
                        new_raccs.append(raccs[i] + t)
                    colsum[pl.ds(off, L)] = colacc
                    return s1_, cnt_, tuple(new_raccs)
                return jc_body

            for base in (0, ROWS // 2):
                s1, cnt, raccs = lax.fori_loop(
                    0, NJC, make_jc_body(base),
                    (s1, cnt, (zero_v,) * (ROWS // 2)))
                for i in range(ROWS // 2):
                    rowvec[base + i] = raccs[i]

            rowsum = plsc.load_gather(
                rowvec, [iota16, jnp.full((L,), 0, jnp.int32)])
            for c in range(1, L):
                rowsum = rowsum + plsc.load_gather(
                    rowvec, [iota16, jnp.full((L,), c, jnp.int32)])
            fvec = jnp.where(rowsum <= 0.001, 1.0, 0.0)
            lc = plsc.load_gather(
                sc_buf, [iota16, jnp.full((L,), N, jnp.int32)])
            return s1, cnt, n0c + fvec, n0d + fvec * lc

        def pair_body(p, carry):
            s1, cnt, n0c, n0d = carry
            g0 = 2 * p
            iou_copy(g0, iou0, sem_i0).wait()
            sc_copy(g0, sc0, sem_s0).wait()
            s1, cnt, n0c, n0d = chunk_compute(g0, iou0, sc0, s1, cnt, n0c, n0d)
            ge = jnp.minimum(g0 + 2, NCHUNK - 1)
            iou_copy(ge, iou0, sem_i0).start()
            sc_copy(ge, sc0, sem_s0).start()

            g1 = 2 * p + 1
            iou_copy(g1, iou1, sem_i1).wait()
            sc_copy(g1, sc1, sem_s1).wait()
            s1, cnt, n0c, n0d = chunk_compute(g1, iou1, sc1, s1, cnt, n0c, n0d)
            go = jnp.minimum(g1 + 2, NCHUNK - 1)
            iou_copy(go, iou1, sem_i1).start()
            sc_copy(go, sc1, sem_s1).start()
            return s1, cnt, n0c, n0d

        zero_v4 = jnp.zeros((L,), jnp.float32)
        s1, cnt, n0c, n0d = lax.fori_loop(
            0, NPAIR, pair_body, (zero_v4, zero_v4, zero_v4, zero_v4))

        iou_copy(NCHUNK - 1, iou0, sem_i0).wait()
        sc_copy(NCHUNK - 1, sc0, sem_s0).wait()
        iou_copy(NCHUNK - 1, iou1, sem_i1).wait()
        sc_copy(NCHUNK - 1, sc1, sem_s1).wait()

        def neg1_body(jc, c):
            n1c_, n1d_ = c
            off = jc * L
            v = colsum[pl.ds(off, L)]
            lr = lastrow[0, pl.ds(off, L)]
            n1 = jnp.where(v <= 0.001, 1.0, 0.0)
            return n1c_ + n1, n1d_ + n1 * lr
        n1c, n1d = lax.fori_loop(0, NJC, neg1_body, (zero_v, zero_v))

        s1s = jnp.full((L,), jnp.sum(s1), jnp.float32)
        cnts = jnp.full((L,), jnp.sum(cnt), jnp.float32)
        n0cs = jnp.full((L,), jnp.sum(n0c), jnp.float32)
        n0ds = jnp.full((L,), jnp.sum(n0d), jnp.float32)
        n1cs = jnp.full((L,), jnp.sum(n1c), jnp.float32)
        n1ds = jnp.full((L,), jnp.sum(n1d), jnp.float32)

        nll_pos = -(ALPHA * s1s) / jnp.maximum(cnts, 1.0)
        nll_neg = (-n0ds - n1ds) / (
            jnp.maximum(n0cs, 1.0) + jnp.maximum(n1cs, 1.0))
        loss = (nll_pos + NEG_WEIGHT * nll_neg) * (1.0 / B)

        out_buf[...] = loss
        pltpu.sync_copy(out_buf, out_hbm.at[b])


def _tc_body(sc_ref, iou_ref, lr_ref, out_ref, colsum, prodsum, cntsum, accs):
    r = pl.program_id(1)
    nr = pl.num_programs(1)

    @pl.when(r == 0)
    def _():
        accs[0] = 0.0
        accs[1] = 0.0
        colsum[...] = jnp.zeros_like(colsum)
        prodsum[...] = jnp.zeros_like(prodsum)
        cntsum[...] = jnp.zeros_like(cntsum)

    sc = sc_ref[0]
    io = iou_ref[0]
    m = io >= MIN_IOU
    t = jnp.where(m, jnp.minimum(io, 1.0), 0.0)
    prod = sc[:, :N] * t
    ones_r = jnp.ones((1, R_TC), jnp.float32)
    colsum[...] = colsum[...] + jnp.dot(
        ones_r, t, preferred_element_type=jnp.float32)
    prodsum[...] = prodsum[...] + jnp.dot(
        ones_r, prod, preferred_element_type=jnp.float32)
    cntsum[...] = cntsum[...] + jnp.dot(
        ones_r, m.astype(jnp.float32), preferred_element_type=jnp.float32)
    rowsum = jnp.dot(t, jnp.ones((N, 1), jnp.float32),
                     preferred_element_type=jnp.float32)
    f = jnp.where(rowsum <= 0.001, 1.0, 0.0)
    accs[0] = accs[0] + jnp.sum(f)
    accs[1] = accs[1] + jnp.sum(f * sc[:, N:N + 1])

    @pl.when(r == nr - 1)
    def _():
        cs = colsum[...]
        n1 = jnp.where(cs <= 0.001, 1.0, 0.0)
        lr = lr_ref[0, 0:1, :N]
        n1c = jnp.sum(n1)
        n1d = jnp.sum(n1 * lr)
        nll_pos = -(ALPHA * jnp.sum(prodsum[...])) / jnp.maximum(
            jnp.sum(cntsum[...]), 1.0)
        nll_neg = (-accs[1] - n1d) / (
            jnp.maximum(accs[0], 1.0) + jnp.maximum(n1c, 1.0))
        loss = (nll_pos + NEG_WEIGHT * nll_neg) * (1.0 / B)
        out_ref[...] = jnp.reshape(loss, (1, 1, 1))


@jax.jit
def _run(scores, ious):
    mesh = plsc.VectorSubcoreMesh(core_axis_name="c", subcore_axis_name="s")
    sc_fn = pl.kernel(
        _sc_body,
        out_type=jax.ShapeDtypeStruct((S_SC, L), jnp.float32),
        mesh=mesh,
        scratch_types=[
            pltpu.VMEM((ROWS, N), jnp.float32),
            pltpu.VMEM((ROWS, N), jnp.float32),
            pltpu.VMEM((ROWS, N + 1), jnp.float32),
            pltpu.VMEM((ROWS, N + 1), jnp.float32),
            pltpu.VMEM((1, N + 1), jnp.float32),
            pltpu.VMEM((N,), jnp.float32),
            pltpu.VMEM((ROWS, L), jnp.float32),
            pltpu.VMEM((L,), jnp.float32),
            pltpu.SemaphoreType.DMA,
            pltpu.SemaphoreType.DMA,
            pltpu.SemaphoreType.DMA,
            pltpu.SemaphoreType.DMA,
        ],
        compiler_params=pltpu.CompilerParams(
            use_tc_tiling_on_sc=True, needs_layout_passes=False),
    )
    sc_out = sc_fn(scores[:S_SC], ious)

    bt = B - S_SC
    scores_tc = scores[S_SC:]
    tc_fn = pl.pallas_call(
        _tc_body,
        grid=(bt, M // R_TC),
        in_specs=[
            pl.BlockSpec((1, R_TC, N + 1), lambda s, r: (s, r, 0)),
            pl.BlockSpec((1, R_TC, N), lambda s, r: (s + S_SC, r, 0)),
            pl.BlockSpec((1, 8, N + 1), lambda s, r: (s, M // 8, 0)),
        ],
        out_specs=pl.BlockSpec((1, 1, 1), lambda s, r: (s, 0, 0)),
        out_shape=jax.ShapeDtypeStruct((bt, 1, 1), jnp.float32),
        scratch_shapes=[
            pltpu.VMEM((1, N), jnp.float32),
            pltpu.VMEM((1, N), jnp.float32),
            pltpu.VMEM((1, N), jnp.float32),
            pltpu.SMEM((2,), jnp.float32),
        ],
    )
    tc_out = tc_fn(scores_tc, ious, scores_tc)

    return jnp.sum(sc_out[:, 0]) + jnp.sum(tc_out)


def kernel(logmax_scores, instance_ious, instance_matches):
    del instance_matches
    return _run(logmax_scores, instance_ious)


# speedup vs baseline: 1.5642x; 1.1838x over previous
import functools

import jax
import jax.numpy as jnp
from jax import lax
from jax.experimental import pallas as pl
from jax.experimental.pallas import tpu as pltpu
from jax.experimental.pallas import tpu_sc as plsc

ALPHA = 2.0
NEG_WEIGHT = 1.0
MIN_IOU = 0.05

L = 16
ROWS = 16
B, M, N = 32, 1024, 1024
S_SC = 16
NCHUNK = M // ROWS
NJC = N // L
NPAIR = NCHUNK // 2
R_TC = 256


def _sc_body(scores_hbm, ious_hbm, out_hbm,
             iou0, iou1, sc0, sc1, lastrow, colsum, rowvec, out_buf,
             sem_i0, sem_i1, sem_s0, sem_s1):
    cid = lax.axis_index("c")
    sid = lax.axis_index("s")
    b = sid * 2 + cid

    @pl.when(b < S_SC)
    def _():
        def iou_copy(g, buf, sem):
            return pltpu.make_async_copy(
                ious_hbm.at[b, pl.ds(g * ROWS, ROWS), pl.ds(0, N)], buf, sem)

        def sc_copy(g, buf, sem):
            return pltpu.make_async_copy(
                scores_hbm.at[b, pl.ds(g * ROWS, ROWS)], buf, sem)

        iou_copy(0, iou0, sem_i0).start()
        sc_copy(0, sc0, sem_s0).start()
        iou_copy(1, iou1, sem_i1).start()
        sc_copy(1, sc1, sem_s1).start()

        pltpu.sync_copy(scores_hbm.at[b, pl.ds(M, 1)], lastrow)

        def zbody(j, _):
            colsum[pl.ds(j * L, L)] = jnp.zeros((L,), jnp.float32)
            return 0
        lax.fori_loop(0, NJC, zbody, 0)

        zero_v = jnp.zeros((L,), jnp.float32)
        iota16 = jnp.arange(L, dtype=jnp.int32)

        def chunk_compute(g, iou_buf, sc_buf, s1, cnt, n0c, n0d):
            def make_jc_body(base):
                def jc_body(jc, c):
                    s1_, cnt_, raccs = c
                    off = jc * L
                    colacc = colsum[pl.ds(off, L)]
                    new_raccs = []
                    for i in range(ROWS // 2):
                        vio = iou_buf[base + i, pl.ds(off, L)]
                        vsc = sc_buf[base + i, pl.ds(off, L)]
                        m = vio >= MIN_IOU
                        t = jnp.where(m, jnp.minimum(vio, 1.0), 0.0)
                        s1_ = s1_ + vsc * t
                        cnt_ = cnt_ + jnp.where(m, 1.0, 0.0)
                        colacc = colacc + t
                        new_raccs.append(raccs[i] + t)
                    colsum[pl.ds(off, L)] = colacc
                    return s1_, cnt_, tuple(new_raccs)
                return jc_body

            for base in (0, ROWS // 2):
                s1, cnt, raccs = lax.fori_loop(
                    0, NJC, make_jc_body(base),
                    (s1, cnt, (zero_v,) * (ROWS // 2)))
                for i in range(ROWS // 2):
                    rowvec[base + i] = raccs[i]

            rowsum = plsc.load_gather(
                rowvec, [iota16, jnp.full((L,), 0, jnp.int32)])
            for c in range(1, L):
                rowsum = rowsum + plsc.load_gather(
                    rowvec, [iota16, jnp.full((L,), c, jnp.int32)])
            fvec = jnp.where(rowsum <= 0.001, 1.0, 0.0)
            lc = plsc.load_gather(
                sc_buf, [iota16, jnp.full((L,), N, jnp.int32)])
            return s1, cnt, n0c + fvec, n0d + fvec * lc

        def pair_body(p, carry):
            s1, cnt, n0c, n0d = carry
            g0 = 2 * p
            iou_copy(g0, iou0, sem_i0).wait()
            sc_copy(g0, sc0, sem_s0).wait()
            s1, cnt, n0c, n0d = chunk_compute(g0, iou0, sc0, s1, cnt, n0c, n0d)
            ge = jnp.minimum(g0 + 2, NCHUNK - 1)
            iou_copy(ge, iou0, sem_i0).start()
            sc_copy(ge, sc0, sem_s0).start()

            g1 = 2 * p + 1
            iou_copy(g1, iou1, sem_i1).wait()
            sc_copy(g1, sc1, sem_s1).wait()
            s1, cnt, n0c, n0d = chunk_compute(g1, iou1, sc1, s1, cnt, n0c, n0d)
            go = jnp.minimum(g1 + 2, NCHUNK - 1)
            iou_copy(go, iou1, sem_i1).start()
            sc_copy(go, sc1, sem_s1).start()
            return s1, cnt, n0c, n0d

        zero_v4 = jnp.zeros((L,), jnp.float32)
        s1, cnt, n0c, n0d = lax.fori_loop(
            0, NPAIR, pair_body, (zero_v4, zero_v4, zero_v4, zero_v4))

        iou_copy(NCHUNK - 1, iou0, sem_i0).wait()
        sc_copy(NCHUNK - 1, sc0, sem_s0).wait()
        iou_copy(NCHUNK - 1, iou1, sem_i1).wait()
        sc_copy(NCHUNK - 1, sc1, sem_s1).wait()

        def neg1_body(jc, c):
            n1c_, n1d_ = c
            off = jc * L
            v = colsum[pl.ds(off, L)]
            lr = lastrow[0, pl.ds(off, L)]
            n1 = jnp.where(v <= 0.001, 1.0, 0.0)
            return n1c_ + n1, n1d_ + n1 * lr
        n1c, n1d = lax.fori_loop(0, NJC, neg1_body, (zero_v, zero_v))

        s1s = jnp.full((L,), jnp.sum(s1), jnp.float32)
        cnts = jnp.full((L,), jnp.sum(cnt), jnp.float32)
        n0cs = jnp.full((L,), jnp.sum(n0c), jnp.float32)
        n0ds = jnp.full((L,), jnp.sum(n0d), jnp.float32)
        n1cs = jnp.full((L,), jnp.sum(n1c), jnp.float32)
        n1ds = jnp.full((L,), jnp.sum(n1d), jnp.float32)

        nll_pos = -(ALPHA * s1s) / jnp.maximum(cnts, 1.0)
        nll_neg = (-n0ds - n1ds) / (
            jnp.maximum(n0cs, 1.0) + jnp.maximum(n1cs, 1.0))
        loss = (nll_pos + NEG_WEIGHT * nll_neg) * (1.0 / B)

        out_buf[...] = loss
        pltpu.sync_copy(out_buf, out_hbm.at[b])


def _tc_body(sc_ref, iou_ref, lr_ref, out_ref, colsum, prodsum, cntsum, accs):
    r = pl.program_id(1)
    nr = pl.num_programs(1)

    @pl.when(r == 0)
    def _():
        accs[0] = 0.0
        accs[1] = 0.0
        colsum[...] = jnp.zeros_like(colsum)
        prodsum[...] = jnp.zeros_like(prodsum)
        cntsum[...] = jnp.zeros_like(cntsum)

    sc = sc_ref[0]
    io = iou_ref[0]
    m = io >= MIN_IOU
    t = jnp.where(m, jnp.minimum(io, 1.0), 0.0)
    prod = sc[:, :N] * t
    ones_r = jnp.ones((1, R_TC), jnp.float32)
    colsum[...] = colsum[...] + jnp.dot(
        ones_r, t, preferred_element_type=jnp.float32)
    prodsum[...] = prodsum[...] + jnp.dot(
        ones_r, prod, preferred_element_type=jnp.float32)
    cntsum[...] = cntsum[...] + jnp.dot(
        ones_r, m.astype(jnp.float32), preferred_element_type=jnp.float32)
    rowsum = jnp.dot(t, jnp.ones((N, 1), jnp.float32),
                     preferred_element_type=jnp.float32)
    f = jnp.where(rowsum <= 0.001, 1.0, 0.0)
    accs[0] = accs[0] + jnp.sum(f)
    accs[1] = accs[1] + jnp.sum(f * sc[:, N:N + 1])

    @pl.when(r == nr - 1)
    def _():
        cs = colsum[...]
        n1 = jnp.where(cs <= 0.001, 1.0, 0.0)
        lr = lr_ref[0, 0:1, :N]
        n1c = jnp.sum(n1)
        n1d = jnp.sum(n1 * lr)
        nll_pos = -(ALPHA * jnp.sum(prodsum[...])) / jnp.maximum(
            jnp.sum(cntsum[...]), 1.0)
        nll_neg = (-accs[1] - n1d) / (
            jnp.maximum(accs[0], 1.0) + jnp.maximum(n1c, 1.0))
        loss = (nll_pos + NEG_WEIGHT * nll_neg) * (1.0 / B)
        out_ref[...] = jnp.reshape(loss, (1, 1, 1))


@jax.jit
def _run(scores, ious):
    mesh = plsc.VectorSubcoreMesh(core_axis_name="c", subcore_axis_name="s")
    sc_fn = pl.kernel(
        _sc_body,
        out_type=jax.ShapeDtypeStruct((S_SC, L), jnp.float32),
        mesh=mesh,
        scratch_types=[
            pltpu.VMEM((ROWS, N), jnp.float32),
            pltpu.VMEM((ROWS, N), jnp.float32),
            pltpu.VMEM((ROWS, N + 1), jnp.float32),
            pltpu.VMEM((ROWS, N + 1), jnp.float32),
            pltpu.VMEM((1, N + 1), jnp.float32),
            pltpu.VMEM((N,), jnp.float32),
            pltpu.VMEM((ROWS, L), jnp.float32),
            pltpu.VMEM((L,), jnp.float32),
            pltpu.SemaphoreType.DMA,
            pltpu.SemaphoreType.DMA,
            pltpu.SemaphoreType.DMA,
            pltpu.SemaphoreType.DMA,
        ],
        compiler_params=pltpu.CompilerParams(
            use_tc_tiling_on_sc=True, needs_layout_passes=False),
    )
    sc_out = sc_fn(scores[:S_SC], ious)

    bt = B - S_SC
    scores_tc = scores[S_SC:]
    tc_fn = pl.pallas_call(
        _tc_body,
        grid=(bt, M // R_TC),
        in_specs=[
            pl.BlockSpec((1, R_TC, N + 1), lambda s, r: (s, r, 0)),
            pl.BlockSpec((1, R_TC, N), lambda s, r: (s + S_SC, r, 0)),
            pl.BlockSpec((1, 8, N + 1), lambda s, r: (s, M // 8, 0)),
        ],
        out_specs=pl.BlockSpec((1, 1, 1), lambda s, r: (s, 0, 0)),
        out_shape=jax.ShapeDtypeStruct((bt, 1, 1), jnp.float32),
        scratch_shapes=[
            pltpu.VMEM((1, N), jnp.float32),
            pltpu.VMEM((1, N), jnp.float32),
            pltpu.VMEM((1, N), jnp.float32),
            pltpu.SMEM((2,), jnp.float32),
        ],
    )
    tc_out = tc_fn(scores_tc, ious, scores_tc)

    return jnp.sum(sc_out[:, 0]) + jnp.sum(tc_out)


def kernel(logmax_scores, instance_ious, instance_matches):
    del instance_matches
    return _run(logmax_scores, instance_ious)


# speedup vs baseline: 2.3499x; 1.5023x over previous
import functools

import jax
import jax.numpy as jnp
from jax import lax
from jax.experimental import pallas as pl
from jax.experimental.pallas import tpu as pltpu
from jax.experimental.pallas import tpu_sc as plsc

ALPHA = 2.0
NEG_WEIGHT = 1.0
MIN_IOU = 0.05

L = 16
ROWS = 16
B, M, N = 32, 1024, 1024
S_SC = 16
HALF_ROWS = M // 2
NCHUNK = HALF_ROWS // ROWS
NJC = N // L
NPAIR = NCHUNK // 2
EX_W = N + L
R_TC = 256


def _sc_body(scores_hbm, ious_hbm, out_hbm,
             iou0, iou1, sc0, sc1, lastrow, colsum, partner, rowvec, out_buf,
             shared, sem_i0, sem_i1, sem_s0, sem_s1):
    cid = lax.axis_index("c")
    sid = lax.axis_index("s")
    b = cid * 8 + sid // 2
    half = sid % 2
    r_base = half * HALF_ROWS

    def iou_copy(g, buf, sem):
        return pltpu.make_async_copy(
            ious_hbm.at[b, pl.ds(r_base + g * ROWS, ROWS), pl.ds(0, N)],
            buf, sem)

    def sc_copy(g, buf, sem):
        return pltpu.make_async_copy(
            scores_hbm.at[b, pl.ds(r_base + g * ROWS, ROWS)], buf, sem)

    iou_copy(0, iou0, sem_i0).start()
    sc_copy(0, sc0, sem_s0).start()
    iou_copy(1, iou1, sem_i1).start()
    sc_copy(1, sc1, sem_s1).start()

    pltpu.sync_copy(scores_hbm.at[b, pl.ds(M, 1)], lastrow)

    def zbody(j, _):
        colsum[0, pl.ds(j * L, L)] = jnp.zeros((L,), jnp.float32)
        return 0
    lax.fori_loop(0, EX_W // L, zbody, 0)

    zero_v = jnp.zeros((L,), jnp.float32)
    iota16 = jnp.arange(L, dtype=jnp.int32)

    def chunk_compute(g, iou_buf, sc_buf, s1, cnt, n0c, n0d):
        def make_jc_body(base):
            def jc_body(jc, c):
                s1_, cnt_, raccs = c
                off = jc * L
                colacc = colsum[0, pl.ds(off, L)]
                new_raccs = []
                for i in range(ROWS // 2):
                    vio = iou_buf[base + i, pl.ds(off, L)]
                    vsc = sc_buf[base + i, pl.ds(off, L)]
                    m = vio >= MIN_IOU
                    t = jnp.where(m, jnp.minimum(vio, 1.0), 0.0)
                    s1_ = s1_ + vsc * t
                    cnt_ = cnt_ + jnp.where(m, 1.0, 0.0)
                    colacc = colacc + t
                    new_raccs.append(raccs[i] + t)
                colsum[0, pl.ds(off, L)] = colacc
                return s1_, cnt_, tuple(new_raccs)
            return jc_body

        for base in (0, ROWS // 2):
            s1, cnt, raccs = lax.fori_loop(
                0, NJC, make_jc_body(base),
                (s1, cnt, (zero_v,) * (ROWS // 2)))
            for i in range(ROWS // 2):
                rowvec[base + i] = raccs[i]

        rowsum = plsc.load_gather(
            rowvec, [iota16, jnp.full((L,), 0, jnp.int32)])
        for c in range(1, L):
            rowsum = rowsum + plsc.load_gather(
                rowvec, [iota16, jnp.full((L,), c, jnp.int32)])
        fvec = jnp.where(rowsum <= 0.001, 1.0, 0.0)
        lc = plsc.load_gather(
            sc_buf, [iota16, jnp.full((L,), N, jnp.int32)])
        return s1, cnt, n0c + fvec, n0d + fvec * lc

    def pair_body(p, carry):
        s1, cnt, n0c, n0d = carry
        g0 = 2 * p
        iou_copy(g0, iou0, sem_i0).wait()
        sc_copy(g0, sc0, sem_s0).wait()
        s1, cnt, n0c, n0d = chunk_compute(g0, iou0, sc0, s1, cnt, n0c, n0d)
        ge = jnp.minimum(g0 + 2, NCHUNK - 1)
        iou_copy(ge, iou0, sem_i0).start()
        sc_copy(ge, sc0, sem_s0).start()

        g1 = 2 * p + 1
        iou_copy(g1, iou1, sem_i1).wait()
        sc_copy(g1, sc1, sem_s1).wait()
        s1, cnt, n0c, n0d = chunk_compute(g1, iou1, sc1, s1, cnt, n0c, n0d)
        go = jnp.minimum(g1 + 2, NCHUNK - 1)
        iou_copy(go, iou1, sem_i1).start()
        sc_copy(go, sc1, sem_s1).start()
        return s1, cnt, n0c, n0d

    s1, cnt, n0c, n0d = lax.fori_loop(
        0, NPAIR, pair_body, (zero_v, zero_v, zero_v, zero_v))

    iou_copy(NCHUNK - 1, iou0, sem_i0).wait()
    sc_copy(NCHUNK - 1, sc0, sem_s0).wait()
    iou_copy(NCHUNK - 1, iou1, sem_i1).wait()
    sc_copy(NCHUNK - 1, sc1, sem_s1).wait()

    s1s = jnp.full((L,), jnp.sum(s1), jnp.float32)
    cnts = jnp.full((L,), jnp.sum(cnt), jnp.float32)
    n0cs = jnp.full((L,), jnp.sum(n0c), jnp.float32)
    n0ds = jnp.full((L,), jnp.sum(n0d), jnp.float32)
    summary = jnp.where(
        iota16 == 0, s1s,
        jnp.where(iota16 == 1, cnts,
                  jnp.where(iota16 == 2, n0cs,
                            jnp.where(iota16 == 3, n0ds, 0.0))))
    colsum[0, pl.ds(N, L)] = summary
    pltpu.sync_copy(colsum, shared.at[sid])
    plsc.subcore_barrier()

    @pl.when(half == 0)
    def _():
        pltpu.sync_copy(shared.at[sid + 1], partner)

        def neg1_body(jc, c):
            n1c_, n1d_ = c
            off = jc * L
            v = colsum[0, pl.ds(off, L)] + partner[0, pl.ds(off, L)]
            lr = lastrow[0, pl.ds(off, L)]
            n1 = jnp.where(v <= 0.001, 1.0, 0.0)
            return n1c_ + n1, n1d_ + n1 * lr
        n1c, n1d = lax.fori_loop(0, NJC, neg1_body, (zero_v, zero_v))

        psv = partner[0, pl.ds(N, L)]
        s1_t = s1s + psv[0]
        cnt_t = cnts + psv[1]
        n0c_t = n0cs + psv[2]
        n0d_t = n0ds + psv[3]
        n1cs = jnp.full((L,), jnp.sum(n1c), jnp.float32)
        n1ds = jnp.full((L,), jnp.sum(n1d), jnp.float32)

        nll_pos = -(ALPHA * s1_t) / jnp.maximum(cnt_t, 1.0)
        nll_neg = (-n0d_t - n1ds) / (
            jnp.maximum(n0c_t, 1.0) + jnp.maximum(n1cs, 1.0))
        loss = (nll_pos + NEG_WEIGHT * nll_neg) * (1.0 / B)

        out_buf[...] = loss
        pltpu.sync_copy(out_buf, out_hbm.at[b])


def _tc_body(sc_ref, iou_ref, lr_ref, out_ref, colsum, prodsum, cntsum, accs):
    r = pl.program_id(1)
    nr = pl.num_programs(1)

    @pl.when(r == 0)
    def _():
        accs[0] = 0.0
        accs[1] = 0.0
        colsum[...] = jnp.zeros_like(colsum)
        prodsum[...] = jnp.zeros_like(prodsum)
        cntsum[...] = jnp.zeros_like(cntsum)

    sc = sc_ref[0]
    io = iou_ref[0]
    m = io >= MIN_IOU
    t = jnp.where(m, jnp.minimum(io, 1.0), 0.0)
    prod = sc[:, :N] * t
    ones_r = jnp.ones((1, R_TC), jnp.float32)
    colsum[...] = colsum[...] + jnp.dot(
        ones_r, t, preferred_element_type=jnp.float32)
    prodsum[...] = prodsum[...] + jnp.dot(
        ones_r, prod, preferred_element_type=jnp.float32)
    cntsum[...] = cntsum[...] + jnp.dot(
        ones_r, m.astype(jnp.float32), preferred_element_type=jnp.float32)
    rowsum = jnp.dot(t, jnp.ones((N, 1), jnp.float32),
                     preferred_element_type=jnp.float32)
    f = jnp.where(rowsum <= 0.001, 1.0, 0.0)
    accs[0] = accs[0] + jnp.sum(f)
    accs[1] = accs[1] + jnp.sum(f * sc[:, N:N + 1])

    @pl.when(r == nr - 1)
    def _():
        cs = colsum[...]
        n1 = jnp.where(cs <= 0.001, 1.0, 0.0)
        lr = lr_ref[0, 0:1, :N]
        n1c = jnp.sum(n1)
        n1d = jnp.sum(n1 * lr)
        nll_pos = -(ALPHA * jnp.sum(prodsum[...])) / jnp.maximum(
            jnp.sum(cntsum[...]), 1.0)
        nll_neg = (-accs[1] - n1d) / (
            jnp.maximum(accs[0], 1.0) + jnp.maximum(n1c, 1.0))
        loss = (nll_pos + NEG_WEIGHT * nll_neg) * (1.0 / B)
        out_ref[...] = jnp.reshape(loss, (1, 1, 1))


@jax.jit
def _run(scores, ious):
    mesh = plsc.VectorSubcoreMesh(core_axis_name="c", subcore_axis_name="s")
    sc_fn = pl.kernel(
        _sc_body,
        out_type=jax.ShapeDtypeStruct((S_SC, L), jnp.float32),
        mesh=mesh,
        scratch_types=[
            pltpu.VMEM((ROWS, N), jnp.float32),
            pltpu.VMEM((ROWS, N), jnp.float32),
            pltpu.VMEM((ROWS, N + 1), jnp.float32),
            pltpu.VMEM((ROWS, N + 1), jnp.float32),
            pltpu.VMEM((1, N + 1), jnp.float32),
            pltpu.VMEM((1, EX_W), jnp.float32),
            pltpu.VMEM((1, EX_W), jnp.float32),
            pltpu.VMEM((ROWS, L), jnp.float32),
            pltpu.VMEM((L,), jnp.float32),
            pltpu.VMEM_SHARED((16, 1, EX_W), jnp.float32),
            pltpu.SemaphoreType.DMA,
            pltpu.SemaphoreType.DMA,
            pltpu.SemaphoreType.DMA,
            pltpu.SemaphoreType.DMA,
        ],
        compiler_params=pltpu.CompilerParams(
            use_tc_tiling_on_sc=True, needs_layout_passes=False),
    )
    sc_out = sc_fn(scores, ious)

    bt = B - S_SC
    tc_fn = pl.pallas_call(
        _tc_body,
        grid=(bt, M // R_TC),
        in_specs=[
            pl.BlockSpec((1, R_TC, N + 1), lambda s, r: (s + S_SC, r, 0)),
            pl.BlockSpec((1, R_TC, N), lambda s, r: (s + S_SC, r, 0)),
            pl.BlockSpec((1, 8, N + 1), lambda s, r: (s + S_SC, M // 8, 0)),
        ],
        out_specs=pl.BlockSpec((1, 1, 1), lambda s, r: (s, 0, 0)),
        out_shape=jax.ShapeDtypeStruct((bt, 1, 1), jnp.float32),
        scratch_shapes=[
            pltpu.VMEM((1, N), jnp.float32),
            pltpu.VMEM((1, N), jnp.float32),
            pltpu.VMEM((1, N), jnp.float32),
            pltpu.SMEM((2,), jnp.float32),
        ],
    )
    tc_out = tc_fn(scores, ious, scores)

    return jnp.sum(sc_out[:, 0]) + jnp.sum(tc_out)


def kernel(logmax_scores, instance_ious, instance_matches):
    del instance_matches
    return _run(logmax_scores, instance_ious)


# speedup vs baseline: 2.4616x; 1.0475x over previous
import functools

import jax
import jax.numpy as jnp
from jax import lax
from jax.experimental import pallas as pl
from jax.experimental.pallas import tpu as pltpu
from jax.experimental.pallas import tpu_sc as plsc

ALPHA = 2.0
NEG_WEIGHT = 1.0
MIN_IOU = 0.05

L = 16
ROWS = 16
B, M, N = 32, 1024, 1024
S_SC = 16
HALF_ROWS = M // 2
NCHUNK = HALF_ROWS // ROWS
NJC = N // L
NPAIR = NCHUNK // 2
EX_W = N + L
R_TC = 512


def _sc_body(scores_hbm, ious_hbm, out_hbm,
             iou0, iou1, sc0, sc1, lastrow, colsum, partner, rowvec, out_buf,
             shared, sem_i0, sem_i1, sem_s0, sem_s1):
    cid = lax.axis_index("c")
    sid = lax.axis_index("s")
    b = cid * 8 + sid // 2
    half = sid % 2
    r_base = half * HALF_ROWS

    def iou_copy(g, buf, sem):
        return pltpu.make_async_copy(
            ious_hbm.at[b, pl.ds(r_base + g * ROWS, ROWS), pl.ds(0, N)],
            buf, sem)

    def sc_copy(g, buf, sem):
        return pltpu.make_async_copy(
            scores_hbm.at[b, pl.ds(r_base + g * ROWS, ROWS)], buf, sem)

    iou_copy(0, iou0, sem_i0).start()
    sc_copy(0, sc0, sem_s0).start()
    iou_copy(1, iou1, sem_i1).start()
    sc_copy(1, sc1, sem_s1).start()

    pltpu.sync_copy(scores_hbm.at[b, pl.ds(M, 1)], lastrow)

    def zbody(j, _):
        colsum[0, pl.ds(j * L, L)] = jnp.zeros((L,), jnp.float32)
        return 0
    lax.fori_loop(0, EX_W // L, zbody, 0)

    zero_v = jnp.zeros((L,), jnp.float32)
    iota16 = jnp.arange(L, dtype=jnp.int32)

    def chunk_compute(g, iou_buf, sc_buf, s1, cnt, n0c, n0d):
        def make_jc_body(base):
            def jc_body(jc, c):
                s1_, cnt_, raccs = c
                off = jc * L
                colacc = colsum[0, pl.ds(off, L)]
                new_raccs = []
                for i in range(ROWS // 2):
                    vio = iou_buf[base + i, pl.ds(off, L)]
                    vsc = sc_buf[base + i, pl.ds(off, L)]
                    m = vio >= MIN_IOU
                    t = jnp.where(m, jnp.minimum(vio, 1.0), 0.0)
                    s1_ = s1_ + vsc * t
                    cnt_ = cnt_ + jnp.where(m, 1.0, 0.0)
                    colacc = colacc + t
                    new_raccs.append(raccs[i] + t)
                colsum[0, pl.ds(off, L)] = colacc
                return s1_, cnt_, tuple(new_raccs)
            return jc_body

        for base in (0, ROWS // 2):
            s1, cnt, raccs = lax.fori_loop(
                0, NJC, make_jc_body(base),
                (s1, cnt, (zero_v,) * (ROWS // 2)))
            for i in range(ROWS // 2):
                rowvec[base + i] = raccs[i]

        rowsum = plsc.load_gather(
            rowvec, [iota16, jnp.full((L,), 0, jnp.int32)])
        for c in range(1, L):
            rowsum = rowsum + plsc.load_gather(
                rowvec, [iota16, jnp.full((L,), c, jnp.int32)])
        fvec = jnp.where(rowsum <= 0.001, 1.0, 0.0)
        lc = plsc.load_gather(
            sc_buf, [iota16, jnp.full((L,), N, jnp.int32)])
        return s1, cnt, n0c + fvec, n0d + fvec * lc

    def pair_body(p, carry):
        s1, cnt, n0c, n0d = carry
        g0 = 2 * p
        iou_copy(g0, iou0, sem_i0).wait()
        sc_copy(g0, sc0, sem_s0).wait()
        s1, cnt, n0c, n0d = chunk_compute(g0, iou0, sc0, s1, cnt, n0c, n0d)
        ge = jnp.minimum(g0 + 2, NCHUNK - 1)
        iou_copy(ge, iou0, sem_i0).start()
        sc_copy(ge, sc0, sem_s0).start()

        g1 = 2 * p + 1
        iou_copy(g1, iou1, sem_i1).wait()
        sc_copy(g1, sc1, sem_s1).wait()
        s1, cnt, n0c, n0d = chunk_compute(g1, iou1, sc1, s1, cnt, n0c, n0d)
        go = jnp.minimum(g1 + 2, NCHUNK - 1)
        iou_copy(go, iou1, sem_i1).start()
        sc_copy(go, sc1, sem_s1).start()
        return s1, cnt, n0c, n0d

    s1, cnt, n0c, n0d = lax.fori_loop(
        0, NPAIR, pair_body, (zero_v, zero_v, zero_v, zero_v))

    iou_copy(NCHUNK - 1, iou0, sem_i0).wait()
    sc_copy(NCHUNK - 1, sc0, sem_s0).wait()
    iou_copy(NCHUNK - 1, iou1, sem_i1).wait()
    sc_copy(NCHUNK - 1, sc1, sem_s1).wait()

    s1s = jnp.full((L,), jnp.sum(s1), jnp.float32)
    cnts = jnp.full((L,), jnp.sum(cnt), jnp.float32)
    n0cs = jnp.full((L,), jnp.sum(n0c), jnp.float32)
    n0ds = jnp.full((L,), jnp.sum(n0d), jnp.float32)
    summary = jnp.where(
        iota16 == 0, s1s,
        jnp.where(iota16 == 1, cnts,
                  jnp.where(iota16 == 2, n0cs,
                            jnp.where(iota16 == 3, n0ds, 0.0))))
    colsum[0, pl.ds(N, L)] = summary
    pltpu.sync_copy(colsum, shared.at[sid])
    plsc.subcore_barrier()

    @pl.when(half == 0)
    def _():
        pltpu.sync_copy(shared.at[sid + 1], partner)

        def neg1_body(jc, c):
            n1c_, n1d_ = c
            off = jc * L
            v = colsum[0, pl.ds(off, L)] + partner[0, pl.ds(off, L)]
            lr = lastrow[0, pl.ds(off, L)]
            n1 = jnp.where(v <= 0.001, 1.0, 0.0)
            return n1c_ + n1, n1d_ + n1 * lr
        n1c, n1d = lax.fori_loop(0, NJC, neg1_body, (zero_v, zero_v))

        psv = partner[0, pl.ds(N, L)]
        s1_t = s1s + psv[0]
        cnt_t = cnts + psv[1]
        n0c_t = n0cs + psv[2]
        n0d_t = n0ds + psv[3]
        n1cs = jnp.full((L,), jnp.sum(n1c), jnp.float32)
        n1ds = jnp.full((L,), jnp.sum(n1d), jnp.float32)

        nll_pos = -(ALPHA * s1_t) / jnp.maximum(cnt_t, 1.0)
        nll_neg = (-n0d_t - n1ds) / (
            jnp.maximum(n0c_t, 1.0) + jnp.maximum(n1cs, 1.0))
        loss = (nll_pos + NEG_WEIGHT * nll_neg) * (1.0 / B)

        out_buf[...] = loss
        pltpu.sync_copy(out_buf, out_hbm.at[b])


def _tc_body(sc_ref, iou_ref, lr_ref, out_ref, colsum, prodsum, cntsum, accs):
    r = pl.program_id(1)
    nr = pl.num_programs(1)

    @pl.when(r == 0)
    def _():
        accs[0] = 0.0
        accs[1] = 0.0
        colsum[...] = jnp.zeros_like(colsum)
        prodsum[...] = jnp.zeros_like(prodsum)
        cntsum[...] = jnp.zeros_like(cntsum)

    sc = sc_ref[0]
    io = iou_ref[0]
    m = io >= MIN_IOU
    t = jnp.where(m, jnp.minimum(io, 1.0), 0.0)
    prod = sc[:, :N] * t
    ones_r = jnp.ones((1, R_TC), jnp.float32)
    colsum[...] = colsum[...] + jnp.dot(
        ones_r, t, preferred_element_type=jnp.float32)
    prodsum[...] = prodsum[...] + jnp.dot(
        ones_r, prod, preferred_element_type=jnp.float32)
    cntsum[...] = cntsum[...] + jnp.dot(
        ones_r, m.astype(jnp.float32), preferred_element_type=jnp.float32)
    rowsum = jnp.dot(t, jnp.ones((N, 1), jnp.float32),
                     preferred_element_type=jnp.float32)
    f = jnp.where(rowsum <= 0.001, 1.0, 0.0)
    accs[0] = accs[0] + jnp.sum(f)
    accs[1] = accs[1] + jnp.sum(f * sc[:, N:N + 1])

    @pl.when(r == nr - 1)
    def _():
        cs = colsum[...]
        n1 = jnp.where(cs <= 0.001, 1.0, 0.0)
        lr = lr_ref[0, 0:1, :N]
        n1c = jnp.sum(n1)
        n1d = jnp.sum(n1 * lr)
        nll_pos = -(ALPHA * jnp.sum(prodsum[...])) / jnp.maximum(
            jnp.sum(cntsum[...]), 1.0)
        nll_neg = (-accs[1] - n1d) / (
            jnp.maximum(accs[0], 1.0) + jnp.maximum(n1c, 1.0))
        loss = (nll_pos + NEG_WEIGHT * nll_neg) * (1.0 / B)
        out_ref[...] = jnp.reshape(loss, (1, 1, 1))


@jax.jit
def _run(scores, ious):
    mesh = plsc.VectorSubcoreMesh(core_axis_name="c", subcore_axis_name="s")
    sc_fn = pl.kernel(
        _sc_body,
        out_type=jax.ShapeDtypeStruct((S_SC, L), jnp.float32),
        mesh=mesh,
        scratch_types=[
            pltpu.VMEM((ROWS, N), jnp.float32),
            pltpu.VMEM((ROWS, N), jnp.float32),
            pltpu.VMEM((ROWS, N + 1), jnp.float32),
            pltpu.VMEM((ROWS, N + 1), jnp.float32),
            pltpu.VMEM((1, N + 1), jnp.float32),
            pltpu.VMEM((1, EX_W), jnp.float32),
            pltpu.VMEM((1, EX_W), jnp.float32),
            pltpu.VMEM((ROWS, L), jnp.float32),
            pltpu.VMEM((L,), jnp.float32),
            pltpu.VMEM_SHARED((16, 1, EX_W), jnp.float32),
            pltpu.SemaphoreType.DMA,
            pltpu.SemaphoreType.DMA,
            pltpu.SemaphoreType.DMA,
            pltpu.SemaphoreType.DMA,
        ],
        compiler_params=pltpu.CompilerParams(
            use_tc_tiling_on_sc=True, needs_layout_passes=False),
    )
    sc_out = sc_fn(scores, ious)

    bt = B - S_SC
    tc_fn = pl.pallas_call(
        _tc_body,
        grid=(bt, M // R_TC),
        in_specs=[
            pl.BlockSpec((1, R_TC, N + 1), lambda s, r: (s + S_SC, r, 0)),
            pl.BlockSpec((1, R_TC, N), lambda s, r: (s + S_SC, r, 0)),
            pl.BlockSpec((1, 8, N + 1), lambda s, r: (s + S_SC, M // 8, 0)),
        ],
        out_specs=pl.BlockSpec((1, 1, 1), lambda s, r: (s, 0, 0)),
        out_shape=jax.ShapeDtypeStruct((bt, 1, 1), jnp.float32),
        scratch_shapes=[
            pltpu.VMEM((1, N), jnp.float32),
            pltpu.VMEM((1, N), jnp.float32),
            pltpu.VMEM((1, N), jnp.float32),
            pltpu.SMEM((2,), jnp.float32),
        ],
    )
    tc_out = tc_fn(scores, ious, scores)

    return jnp.sum(sc_out[:, 0]) + jnp.sum(tc_out)


def kernel(logmax_scores, instance_ious, instance_matches):
    del instance_matches
    return _run(logmax_scores, instance_ious)
